# baseline (speedup 1.0000x reference)
import functools

import jax
import jax.numpy as jnp
from jax import lax
from jax.experimental import pallas as pl
from jax.experimental.pallas import tpu as pltpu
from jax.experimental.pallas import tpu_sc as plsc

_DIM = 128
_NC = 2
_NS = 16
_CHUNK = 128
_NBUF = 2
_TRASH = _NS * 128


def _segsum_body(q_hbm, k_hbm, seg_hbm, z_hbm, qs_hbm, ks_hbm,
                 idx_v, qb0, qb1, kb0, kb1, accq_sh, acck_sh,
                 sq0, sq1, sk0, sk1, ssq, ssk):
    c = lax.axis_index("c")
    s = lax.axis_index("s")
    wid = s * _NC + c
    qbufs = (qb0, qb1)
    kbufs = (kb0, kb1)
    qsems = (sq0, sq1)
    ksems = (sk0, sk1)

    pltpu.sync_copy(seg_hbm.at[s], idx_v)
    pltpu.sync_copy(z_hbm, accq_sh.at[pl.ds(s * 128, 128)])
    pltpu.sync_copy(z_hbm, acck_sh.at[pl.ds(s * 128, 128)])

    t = q_hbm.shape[1]
    n_full = t // _CHUNK
    tail_rows = t - n_full * _CHUNK
    full_groups = n_full // _NBUF

    def issue(cc, b):
        pltpu.async_copy(q_hbm.at[wid, pl.ds(cc * _CHUNK, _CHUNK)],
                         qbufs[b], qsems[b])
        pltpu.async_copy(k_hbm.at[wid, pl.ds(cc * _CHUNK, _CHUNK)],
                         kbufs[b], ksems[b])

    for b in range(_NBUF):
        issue(b, b)

    def outer(g, carry):
        for b in range(_NBUF):
            cc = g * _NBUF + b
            pltpu.make_async_copy(q_hbm.at[wid, pl.ds(cc * _CHUNK, _CHUNK)],
                                  qbufs[b], qsems[b]).wait()
            pltpu.make_async_copy(k_hbm.at[wid, pl.ds(cc * _CHUNK, _CHUNK)],
                                  kbufs[b], ksems[b]).wait()
            cpq = pltpu.async_copy(qbufs[b], accq_sh.at[idx_v.at[cc]], ssq,
                                   add=True)
            cpk = pltpu.async_copy(kbufs[b], acck_sh.at[idx_v.at[cc]], ssk,
                                   add=True)
            cpq.wait()
            cpk.wait()
            nxt = cc + _NBUF

            @pl.when(nxt < n_full)
            def _():
                issue(nxt, b)
        return carry

    lax.fori_loop(0, full_groups, outer, 0)
    for b in range(n_full % _NBUF):
        cc = full_groups * _NBUF + b
        pltpu.make_async_copy(q_hbm.at[wid, pl.ds(cc * _CHUNK, _CHUNK)],
                              qbufs[b], qsems[b]).wait()
        pltpu.make_async_copy(k_hbm.at[wid, pl.ds(cc * _CHUNK, _CHUNK)],
                              kbufs[b], ksems[b]).wait()
        pltpu.sync_copy(qbufs[b], accq_sh.at[idx_v.at[cc]], add=True)
        pltpu.sync_copy(kbufs[b], acck_sh.at[idx_v.at[cc]], add=True)
    if tail_rows:
        pltpu.sync_copy(q_hbm.at[wid, pl.ds(n_full * _CHUNK, tail_rows)],
                        qbufs[0].at[pl.ds(0, tail_rows)])
        pltpu.sync_copy(k_hbm.at[wid, pl.ds(n_full * _CHUNK, tail_rows)],
                        kbufs[0].at[pl.ds(0, tail_rows)])
        cpq = pltpu.async_copy(qbufs[0], accq_sh.at[idx_v.at[n_full]], ssq,
                               add=True)
        cpk = pltpu.async_copy(kbufs[0], acck_sh.at[idx_v.at[n_full]], ssk,
                               add=True)
        cpq.wait()
        cpk.wait()

    pltpu.sync_copy(accq_sh.at[pl.ds(s * 128, 128)], qs_hbm.at[wid])
    pltpu.sync_copy(acck_sh.at[pl.ds(s * 128, 128)], ks_hbm.at[wid])


def _segsum_sc(q, k, seg_off, zeros):
    b_h, t, d = q.shape
    L = 128
    mesh = plsc.VectorSubcoreMesh(core_axis_name="c", subcore_axis_name="s")
    f = pl.kernel(
        _segsum_body,
        out_type=[jax.ShapeDtypeStruct((b_h, L, d), jnp.float32),
                  jax.ShapeDtypeStruct((b_h, L, d), jnp.float32)],
        mesh=mesh,
        scratch_types=[
            pltpu.VMEM(((t + _CHUNK - 1) // _CHUNK, _CHUNK), jnp.int32),
            pltpu.VMEM((_CHUNK, d), jnp.float32),
            pltpu.VMEM((_CHUNK, d), jnp.float32),
            pltpu.VMEM((_CHUNK, d), jnp.float32),
            pltpu.VMEM((_CHUNK, d), jnp.float32),
            pltpu.VMEM_SHARED((_NS * L + 8, d), jnp.float32),
            pltpu.VMEM_SHARED((_NS * L + 8, d), jnp.float32),
            pltpu.SemaphoreType.DMA,
            pltpu.SemaphoreType.DMA,
            pltpu.SemaphoreType.DMA,
            pltpu.SemaphoreType.DMA,
            pltpu.SemaphoreType.DMA,
            pltpu.SemaphoreType.DMA,
        ],
    )
    return f(q, k, seg_off, zeros)


def _meta_kernel(sizes_ref, segoff_ref, *, L, n_chunks, t):
    pos = (jax.lax.broadcasted_iota(jnp.int32, (n_chunks, _CHUNK), 0) * _CHUNK
           + jax.lax.broadcasted_iota(jnp.int32, (n_chunks, _CHUNK), 1))
    seg = jnp.zeros((n_chunks, _CHUNK), jnp.int32)
    tot = sizes_ref[0, 0] * 0
    for l in range(L):
        tot = tot + sizes_ref[0, l]
        seg = seg + (pos >= tot).astype(jnp.int32)
    for s in range(_NS):
        segoff_ref[s] = jnp.where(pos < t, seg + s * L, _TRASH)


def _meta_tc(sizes, t):
    num_samples, L = sizes.shape
    n_chunks = (t + _CHUNK - 1) // _CHUNK
    return pl.pallas_call(
        functools.partial(_meta_kernel, L=L, n_chunks=n_chunks, t=t),
        in_specs=[pl.BlockSpec((1, L), lambda: (0, 0))],
        out_specs=pl.BlockSpec((_NS, n_chunks, _CHUNK), lambda: (0, 0, 0)),
        out_shape=jax.ShapeDtypeStruct((_NS, n_chunks, _CHUNK), jnp.int32),
    )(sizes)


def _finish_kernel(topk_ref, qs_ref, ks_ref, out_ref, *, L):
    qs = qs_ref[0].astype(jnp.bfloat16)
    ks = ks_ref[0].astype(jnp.bfloat16)
    R = jax.lax.dot_general(qs, ks, (((1,), (1,)), ((), ())),
                            preferred_element_type=jnp.float32)
    R = R * jnp.float32(_DIM ** -0.5)
    R = R * topk_ref[...]
    m = jnp.max(R, axis=-1, keepdims=True)
    e = jnp.exp(R - m)
    ssum = jnp.sum(e, axis=-1, keepdims=True)
    sm = e / ssum
    msm = jnp.max(sm, axis=-1, keepdims=True)
    lidx = jax.lax.broadcasted_iota(jnp.int32, (L, L), 1)
    jstar = jnp.min(jnp.where(sm >= msm, lidx, L), axis=-1, keepdims=True)
    out_ref[0] = jnp.where(lidx == jstar, msm, 0.0)


def _finish_tc(qs, ks, topk_row):
    b_h, L, d = qs.shape
    return pl.pallas_call(
        functools.partial(_finish_kernel, L=L),
        grid=(b_h,),
        in_specs=[
            pl.BlockSpec((1, L), lambda h: (0, 0)),
            pl.BlockSpec((1, L, d), lambda h: (h, 0, 0)),
            pl.BlockSpec((1, L, d), lambda h: (h, 0, 0)),
        ],
        out_specs=pl.BlockSpec((1, L, L), lambda h: (h, 0, 0)),
        out_shape=jax.ShapeDtypeStruct((b_h, L, L), jnp.float32),
    )(topk_row, qs, ks)


def kernel(q, k, bucket_size, topk):
    b_h, t, d = q.shape
    num_samples, L = bucket_size.shape

    seg_off = _meta_tc(bucket_size.astype(jnp.int32), t)
    zeros = jnp.zeros((L, d), jnp.float32)

    qs, ks = _segsum_sc(q, k, seg_off, zeros)
    topk_row = jnp.full((1, L), topk, dtype=jnp.float32)
    return _finish_tc(qs, ks, topk_row)


# speedup vs baseline: 1.2546x; 1.2546x over previous
import functools

import jax
import jax.numpy as jnp
from jax import lax
from jax.experimental import pallas as pl
from jax.experimental.pallas import tpu as pltpu
from jax.experimental.pallas import tpu_sc as plsc

_DIM = 128
_NC = 2
_NS = 16
_CHUNK = 64
_NBUF = 4
_HBATCH = 8


def _segsum_body(q_hbm, k_hbm, seg_hbm, z_hbm, qs_hbm, ks_hbm,
                 idx_v, b0, b1, b2, b3, accq_sh, acck_sh, s0, s1, s2, s3):
    c = lax.axis_index("c")
    s = lax.axis_index("s")
    wid = s * _NC + c
    bufs = (b0, b1, b2, b3)
    sems = (s0, s1, s2, s3)

    pltpu.sync_copy(seg_hbm.at[s], idx_v)
    pltpu.sync_copy(z_hbm, accq_sh.at[pl.ds(s * 128, 128)])
    pltpu.sync_copy(z_hbm, acck_sh.at[pl.ds(s * 128, 128)])

    n_chunks = q_hbm.shape[1] // _CHUNK
    full_groups = n_chunks // _NBUF
    tail = n_chunks % _NBUF

    def stream_tensor(x_hbm, acc_sh):
        for b in range(_NBUF):
            pltpu.async_copy(x_hbm.at[wid, pl.ds(b * _CHUNK, _CHUNK)],
                             bufs[b], sems[b])

        def outer(g, carry):
            for b in range(_NBUF):
                cc = g * _NBUF + b
                pltpu.make_async_copy(
                    x_hbm.at[wid, pl.ds(cc * _CHUNK, _CHUNK)],
                    bufs[b], sems[b]).wait()
                pltpu.sync_copy(bufs[b], acc_sh.at[idx_v.at[cc]], add=True)
                nxt = cc + _NBUF

                @pl.when(nxt < n_chunks)
                def _():
                    pltpu.async_copy(x_hbm.at[wid, pl.ds(nxt * _CHUNK, _CHUNK)],
                                     bufs[b], sems[b])
            return carry

        lax.fori_loop(0, full_groups, outer, 0)
        for b in range(tail):
            cc = full_groups * _NBUF + b
            pltpu.make_async_copy(
                x_hbm.at[wid, pl.ds(cc * _CHUNK, _CHUNK)],
                bufs[b], sems[b]).wait()
            pltpu.sync_copy(bufs[b], acc_sh.at[idx_v.at[cc]], add=True)

    stream_tensor(q_hbm, accq_sh)
    stream_tensor(k_hbm, acck_sh)

    pltpu.sync_copy(accq_sh.at[pl.ds(s * 128, 128)], qs_hbm.at[wid])
    pltpu.sync_copy(acck_sh.at[pl.ds(s * 128, 128)], ks_hbm.at[wid])


def _segsum_sc(q, k, seg_off, zeros):
    b_h, t, d = q.shape
    L = 128
    mesh = plsc.VectorSubcoreMesh(core_axis_name="c", subcore_axis_name="s")
    f = pl.kernel(
        _segsum_body,
        out_type=[jax.ShapeDtypeStruct((b_h, L, d), jnp.float32),
                  jax.ShapeDtypeStruct((b_h, L, d), jnp.float32)],
        mesh=mesh,
        scratch_types=[
            pltpu.VMEM((t // _CHUNK, _CHUNK), jnp.int32),
            pltpu.VMEM((_CHUNK, d), jnp.float32),
            pltpu.VMEM((_CHUNK, d), jnp.float32),
            pltpu.VMEM((_CHUNK, d), jnp.float32),
            pltpu.VMEM((_CHUNK, d), jnp.float32),
            pltpu.VMEM_SHARED((_NS * L, d), jnp.float32),
            pltpu.VMEM_SHARED((_NS * L, d), jnp.float32),
            pltpu.SemaphoreType.DMA,
            pltpu.SemaphoreType.DMA,
            pltpu.SemaphoreType.DMA,
            pltpu.SemaphoreType.DMA,
        ],
    )
    return f(q, k, seg_off, zeros)


def _meta_kernel(sizes_ref, segoff_ref, *, L, n_chunks):
    pos = (jax.lax.broadcasted_iota(jnp.int32, (n_chunks, _CHUNK), 0) * _CHUNK
           + jax.lax.broadcasted_iota(jnp.int32, (n_chunks, _CHUNK), 1))
    seg = jnp.zeros((n_chunks, _CHUNK), jnp.int32)
    tot = sizes_ref[0, 0] * 0
    for l in range(L):
        tot = tot + sizes_ref[0, l]
        seg = seg + (pos >= tot).astype(jnp.int32)
    for s in range(_NS):
        segoff_ref[s] = seg + s * L


def _meta_tc(sizes, t):
    num_samples, L = sizes.shape
    n_chunks = t // _CHUNK
    return pl.pallas_call(
        functools.partial(_meta_kernel, L=L, n_chunks=n_chunks),
        in_specs=[pl.BlockSpec((1, L), lambda: (0, 0))],
        out_specs=pl.BlockSpec((_NS, n_chunks, _CHUNK), lambda: (0, 0, 0)),
        out_shape=jax.ShapeDtypeStruct((_NS, n_chunks, _CHUNK), jnp.int32),
    )(sizes)


def _finish_kernel(topk_ref, qs_ref, ks_ref, out_ref, *, L):
    for h in range(_HBATCH):
        qs = qs_ref[h].astype(jnp.bfloat16)
        ks = ks_ref[h].astype(jnp.bfloat16)
        R = jax.lax.dot_general(qs, ks, (((1,), (1,)), ((), ())),
                                preferred_element_type=jnp.float32)
        R = R * jnp.float32(_DIM ** -0.5)
        R = R * topk_ref[...]
        m = jnp.max(R, axis=-1, keepdims=True)
        e = jnp.exp(R - m)
        ssum = jnp.sum(e, axis=-1, keepdims=True)
        sm = e / ssum
        msm = jnp.max(sm, axis=-1, keepdims=True)
        lidx = jax.lax.broadcasted_iota(jnp.int32, (L, L), 1)
        jstar = jnp.min(jnp.where(sm >= msm, lidx, L), axis=-1, keepdims=True)
        out_ref[h] = jnp.where(lidx == jstar, msm, 0.0)


def _finish_tc(qs, ks, topk_row):
    b_h, L, d = qs.shape
    return pl.pallas_call(
        functools.partial(_finish_kernel, L=L),
        grid=(b_h // _HBATCH,),
        in_specs=[
            pl.BlockSpec((1, L), lambda g: (0, 0)),
            pl.BlockSpec((_HBATCH, L, d), lambda g: (g, 0, 0)),
            pl.BlockSpec((_HBATCH, L, d), lambda g: (g, 0, 0)),
        ],
        out_specs=pl.BlockSpec((_HBATCH, L, L), lambda g: (g, 0, 0)),
        out_shape=jax.ShapeDtypeStruct((b_h, L, L), jnp.float32),
    )(topk_row, qs, ks)


def kernel(q, k, bucket_size, topk):
    b_h, t, d = q.shape
    num_samples, L = bucket_size.shape

    seg_off = _meta_tc(bucket_size.astype(jnp.int32), t)
    zeros = jnp.zeros((L, d), jnp.float32)

    qs, ks = _segsum_sc(q, k, seg_off, zeros)
    topk_row = jnp.full((1, L), topk, dtype=jnp.float32)
    return _finish_tc(qs, ks, topk_row)
